# baseline (speedup 1.0000x reference)
import functools

import jax
import jax.numpy as jnp
from jax import lax
from jax.experimental import pallas as pl
from jax.experimental.pallas import tpu as pltpu
from jax.experimental.pallas import tpu_sc as plsc

NC = 80


def _corners(cx, cy, w, h):
    x1 = cx - w / 2
    y1 = cy - h / 2
    x2 = cx + w / 2
    y2 = cy + h / 2
    return y1, x1, y2, x2


def _inter_ue(g_cx, g_cy, g_w, g_h, p_cx, p_cy, p_w, p_h):
    gy1, gx1, gy2, gx2 = _corners(g_cx, g_cy, g_w, g_h)
    py1, px1, py2, px2 = _corners(p_cx, p_cy, p_w, p_h)
    in_ymin = jnp.maximum(gy1, py1)
    in_xmin = jnp.maximum(gx1, px1)
    in_ymax = jnp.minimum(gy2, py2)
    in_xmax = jnp.minimum(gx2, px2)
    in_w = jnp.maximum(0.0, in_xmax - in_xmin)
    in_h = jnp.maximum(0.0, in_ymax - in_ymin)
    inter = in_w * in_h
    pa_eps = p_w * p_h + 1e-5
    ue = ((g_w * g_h) + pa_eps) - inter
    return inter, ue


def _split_gt(gt):
    g_cx = gt[:, 0:1]
    g_cy = gt[:, 1:2]
    g_w = gt[:, 2:3]
    g_h = gt[:, 3:4]
    return g_cx, g_cy, g_w, g_h


def _argmax_kernel(gt_ref, px_ref, py_ref, pw_ref, ph_ref, best_ref, *,
                   num_pr):
    gt = gt_ref[0]
    g_cx, g_cy, g_w, g_h = _split_gt(gt)
    inter, ue = _inter_ue(g_cx, g_cy, g_w, g_h,
                          px_ref[0], py_ref[0], pw_ref[0], ph_ref[0])
    iou = inter / ue
    bmax = jnp.max(iou, axis=1, keepdims=True)
    pidx = jax.lax.broadcasted_iota(jnp.int32, iou.shape, 1)
    barg = jnp.min(jnp.where(iou == bmax, pidx, num_pr), axis=1, keepdims=True)
    best_ref[0] = barg


def _assign_kernel(gt_ref, px_ref, py_ref, pw_ref, ph_ref,
                   bx_ref, by_ref, bw_ref, bh_ref, best_ref, packed_ref,
                   *, blk):
    pb = pl.program_id(1)
    gt = gt_ref[0]
    ng = gt.shape[0]
    g_cx, g_cy, g_w, g_h = _split_gt(gt)
    g_cls = gt[:, 4:5]
    g_conf = gt[:, 5:6]

    inter, ue = _inter_ue(g_cx, g_cy, g_w, g_h,
                          px_ref[0], py_ref[0], pw_ref[0], ph_ref[0])
    thr = (inter + inter) >= ue
    ign = (2.5 * inter >= ue) & (~thr)

    pidx = jax.lax.broadcasted_iota(jnp.int32, (1, blk), 1) + pb * blk
    best = best_ref[0]
    best_x = jnp.where(g_conf > 0.0, best, -7)
    is_best = pidx == best_x

    g_iota = jax.lax.broadcasted_iota(jnp.int32, (ng, 1), 0)
    cls_i = g_cls.astype(jnp.int32)
    v_thr = g_iota * 128 + cls_i
    v_best = v_thr + ng * 128
    score = jnp.where(is_best, v_best, jnp.where(thr, v_thr, -1))
    smax = jnp.max(score, axis=0, keepdims=True)
    matched = smax >= 0
    cls_true = jnp.where(matched, smax & 127, NC).astype(jnp.float32)

    cnt = thr.astype(jnp.float32) + is_best.astype(jnp.float32)
    lgw = jnp.log(jnp.where(g_w > 0.0, g_w, 1.0))
    lgh = jnp.log(jnp.where(g_h > 0.0, g_h, 1.0))
    wmat = jnp.transpose(jnp.concatenate(
        [jnp.ones_like(g_cx), g_cx, g_cy, lgw, lgh], axis=1))
    sums = jax.lax.dot_general(
        wmat, cnt, (((1,), (0,)), ((), ())),
        precision=jax.lax.Precision.HIGHEST,
        preferred_element_type=jnp.float32)
    s_cnt = sums[0:1, :]
    s_cx = sums[1:2, :]
    s_cy = sums[2:3, :]
    s_lw = sums[3:4, :]
    s_lh = sums[4:5, :]
    b_cx = bx_ref[0]
    b_cy = by_ref[0]
    b_w = bw_ref[0]
    b_h = bh_ref[0]
    l0 = (s_cx - b_cx * s_cnt) / b_w
    l1 = (s_cy - b_cy * s_cnt) / b_h
    l2 = s_lw - s_cnt * jnp.log(b_w)
    l3 = s_lh - s_cnt * jnp.log(b_h)

    bg = (~matched).astype(jnp.float32)
    ignore_any = jnp.max(ign.astype(jnp.int32), axis=0, keepdims=True) > 0
    amask = jnp.where(ignore_any, -1.0, bg)

    zeros2 = jnp.zeros((2, blk), jnp.float32)
    packed = jnp.concatenate(
        [cls_true, l0, l1, l2, l3, amask, zeros2], axis=0)
    packed_ref[0] = jnp.transpose(packed)


def _sc_emit_kernel(packed_hbm, cls_hbm, loc_hbm, msk_hbm,
                    pk_v, cls_v, loc_v, msk_v, *, npp, n_per_w, chunk):
    wid = lax.axis_index("s") * 2 + lax.axis_index("c")
    n_batch = 20000 // n_per_w
    b = wid // n_batch
    p0 = (wid % n_batch) * n_per_w
    iota = lax.iota(jnp.int32, 16)
    last = n_per_w - 1

    pltpu.sync_copy(
        packed_hbm.at[pl.ds((b * npp + p0) * 8, n_per_w * 8)], pk_v)

    def loc_body(i, carry):
        pe = jnp.minimum(i * 16 + iota, last)
        base8 = pe * 8
        for c in range(4):
            v = plsc.load_gather(pk_v, [base8 + (1 + c)])
            plsc.store_scatter(loc_v, [pe * 4 + c], v)
        mk = plsc.load_gather(pk_v, [base8 + 5])
        plsc.store_scatter(msk_v, [pe], mk)
        return carry

    lax.fori_loop(0, (n_per_w + 15) // 16, loc_body, 0)
    pltpu.sync_copy(loc_v, loc_hbm.at[pl.ds(wid * (n_per_w * 4), n_per_w * 4)])
    pltpu.sync_copy(msk_v, msk_hbm.at[pl.ds(wid * n_per_w, n_per_w)])

    def zero_body(z, carry):
        cls_v[pl.ds(z * 16, 16)] = jnp.zeros((16,), jnp.float32)
        return carry

    lax.fori_loop(0, chunk * NC // 16, zero_body, 0)

    steps = (chunk + 15) // 16

    def make_patch_body(base, matched_value):
        def body(j, carry):
            pe = jnp.minimum(base + j * 16 + iota, base + (chunk - 1))
            lbl = plsc.load_gather(pk_v, [pe * 8]).astype(jnp.int32)
            off = (pe - base) * NC + jnp.minimum(lbl, NC - 1)
            if matched_value:
                val = jnp.where(lbl < NC, 1.0, 0.0).astype(jnp.float32)
            else:
                val = jnp.zeros((16,), jnp.float32)
            plsc.store_scatter(cls_v, [off], val)
            return carry

        return body

    def chunk_body(ch, carry):
        base = ch * chunk
        lax.fori_loop(0, steps, make_patch_body(base, True), 0)
        pltpu.sync_copy(
            cls_v, cls_hbm.at[pl.ds((wid * n_per_w + base) * NC, chunk * NC)])
        lax.fori_loop(0, steps, make_patch_body(base, False), 0)
        return carry

    lax.fori_loop(0, n_per_w // chunk, chunk_body, 0)


@jax.jit
def kernel(gt_boxes, pr_boxes):
    B, NG, _ = gt_boxes.shape
    _, NP, _ = pr_boxes.shape
    blk = 4096
    npad = -NP % blk
    NPP = NP + npad
    pb_steps = NPP // blk

    gt_valid = gt_boxes[:, :, 0:1] != -1.0
    gt_clean = jnp.where(gt_valid, gt_boxes, jnp.zeros((), jnp.float32))
    gt_clean = jnp.concatenate([gt_clean[:, :, :5], gt_boxes[:, :, 5:6]],
                               axis=-1)

    comp = jnp.pad(pr_boxes, ((0, 0), (0, npad), (0, 0)))
    px = comp[:, :, 0].reshape(B, 1, NPP)
    py = comp[:, :, 1].reshape(B, 1, NPP)
    pw = comp[:, :, 2].reshape(B, 1, NPP)
    ph = comp[:, :, 3].reshape(B, 1, NPP)

    full_spec = lambda: pl.BlockSpec((1, 1, NPP), lambda b: (0, 0, 0))
    row_spec = lambda f: pl.BlockSpec((1, 1, blk), f)
    b0 = lambda b, p: (0, 0, p)
    bb = lambda b, p: (b, 0, p)

    best = pl.pallas_call(
        functools.partial(_argmax_kernel, num_pr=NPP),
        grid=(B,),
        in_specs=[
            pl.BlockSpec((1, NG, 6), lambda b: (b, 0, 0)),
            full_spec(), full_spec(), full_spec(), full_spec(),
        ],
        out_specs=pl.BlockSpec((1, NG, 1), lambda b: (b, 0, 0)),
        out_shape=jax.ShapeDtypeStruct((B, NG, 1), jnp.int32),
    )(gt_clean, px, py, pw, ph)

    packed = pl.pallas_call(
        functools.partial(_assign_kernel, blk=blk),
        grid=(B, pb_steps),
        in_specs=[
            pl.BlockSpec((1, NG, 6), lambda b, p: (b, 0, 0)),
            row_spec(b0), row_spec(b0), row_spec(b0), row_spec(b0),
            row_spec(bb), row_spec(bb), row_spec(bb), row_spec(bb),
            pl.BlockSpec((1, NG, 1), lambda b, p: (b, 0, 0)),
        ],
        out_specs=pl.BlockSpec((1, blk, 8), lambda b, p: (b, p, 0)),
        out_shape=jax.ShapeDtypeStruct((B, NPP, 8), jnp.float32),
    )(gt_clean, px, py, pw, ph, px, py, pw, ph, best)

    n_per_w = (B * NP) // 32
    chunk = 500
    mesh = plsc.VectorSubcoreMesh(core_axis_name="c", subcore_axis_name="s")
    sc_emit = pl.kernel(
        functools.partial(_sc_emit_kernel, npp=NPP, n_per_w=n_per_w,
                          chunk=chunk),
        mesh=mesh,
        compiler_params=pltpu.CompilerParams(needs_layout_passes=False),
        out_type=[
            jax.ShapeDtypeStruct((B * NP * NC,), jnp.float32),
            jax.ShapeDtypeStruct((B * NP * 4,), jnp.float32),
            jax.ShapeDtypeStruct((B * NP,), jnp.float32),
        ],
        scratch_types=[
            pltpu.VMEM((n_per_w * 8,), jnp.float32),
            pltpu.VMEM((chunk * NC,), jnp.float32),
            pltpu.VMEM((n_per_w * 4,), jnp.float32),
            pltpu.VMEM((n_per_w,), jnp.float32),
        ],
    )
    cls_out, loc_true, amask = sc_emit(packed.reshape(-1))

    return (cls_out.reshape(B, NP, NC), loc_true.reshape(B, NP, 4),
            amask.reshape(B, NP, 1))


# speedup vs baseline: 1.2193x; 1.2193x over previous
import functools

import jax
import jax.numpy as jnp
from jax.experimental import pallas as pl
from jax.experimental.pallas import tpu as pltpu

NC = 80


def _corners(cx, cy, w, h):
    x1 = cx - w / 2
    y1 = cy - h / 2
    x2 = cx + w / 2
    y2 = cy + h / 2
    return y1, x1, y2, x2


def _inter_ue(g_cx, g_cy, g_w, g_h, p_cx, p_cy, p_w, p_h):
    gy1, gx1, gy2, gx2 = _corners(g_cx, g_cy, g_w, g_h)
    py1, px1, py2, px2 = _corners(p_cx, p_cy, p_w, p_h)
    in_ymin = jnp.maximum(gy1, py1)
    in_xmin = jnp.maximum(gx1, px1)
    in_ymax = jnp.minimum(gy2, py2)
    in_xmax = jnp.minimum(gx2, px2)
    in_w = jnp.maximum(0.0, in_xmax - in_xmin)
    in_h = jnp.maximum(0.0, in_ymax - in_ymin)
    inter = in_w * in_h
    pa_eps = p_w * p_h + 1e-5
    ue = ((g_w * g_h) + pa_eps) - inter
    return inter, ue


def _split_gt(gt):
    g_cx = gt[:, 0:1]
    g_cy = gt[:, 1:2]
    g_w = gt[:, 2:3]
    g_h = gt[:, 3:4]
    return g_cx, g_cy, g_w, g_h


def _argmax_kernel(gt_ref, px_ref, py_ref, pw_ref, ph_ref, best_ref, *,
                   num_pr):
    gt = gt_ref[0]
    g_cx, g_cy, g_w, g_h = _split_gt(gt)
    inter, ue = _inter_ue(g_cx, g_cy, g_w, g_h,
                          px_ref[0], py_ref[0], pw_ref[0], ph_ref[0])
    iou = inter / ue
    bmax = jnp.max(iou, axis=1, keepdims=True)
    pidx = jax.lax.broadcasted_iota(jnp.int32, iou.shape, 1)
    barg = jnp.min(jnp.where(iou == bmax, pidx, num_pr), axis=1, keepdims=True)
    best_ref[0] = barg


def _assign_kernel(gt_ref, px_ref, py_ref, pw_ref, ph_ref,
                   bx_ref, by_ref, bw_ref, bh_ref, best_ref, packed_ref,
                   *, blk):
    pb = pl.program_id(1)
    gt = gt_ref[0]
    ng = gt.shape[0]
    g_cx, g_cy, g_w, g_h = _split_gt(gt)
    g_cls = gt[:, 4:5]
    g_conf = gt[:, 5:6]

    inter, ue = _inter_ue(g_cx, g_cy, g_w, g_h,
                          px_ref[0], py_ref[0], pw_ref[0], ph_ref[0])
    thr = (inter + inter) >= ue
    ign = (2.5 * inter >= ue) & (~thr)

    pidx = jax.lax.broadcasted_iota(jnp.int32, (1, blk), 1) + pb * blk
    best = best_ref[0]
    best_x = jnp.where(g_conf > 0.0, best, -7)
    is_best = pidx == best_x

    g_iota = jax.lax.broadcasted_iota(jnp.int32, (ng, 1), 0)
    cls_i = g_cls.astype(jnp.int32)
    v_thr = g_iota * 128 + cls_i
    v_best = v_thr + ng * 128
    score = jnp.where(is_best, v_best, jnp.where(thr, v_thr, -1))
    smax = jnp.max(score, axis=0, keepdims=True)
    matched = smax >= 0
    cls_true = jnp.where(matched, smax & 127, NC).astype(jnp.float32)

    cnt = (thr.astype(jnp.bfloat16) + is_best.astype(jnp.bfloat16))
    lgw = jnp.log(jnp.where(g_w > 0.0, g_w, 1.0))
    lgh = jnp.log(jnp.where(g_h > 0.0, g_h, 1.0))
    wmat = jnp.transpose(jnp.concatenate(
        [jnp.ones_like(g_cx), g_cx, g_cy, lgw, lgh], axis=1))
    w_hi = wmat.astype(jnp.bfloat16)
    w_lo = (wmat - w_hi.astype(jnp.float32)).astype(jnp.bfloat16)
    dims = (((1,), (0,)), ((), ()))
    sums = (
        jax.lax.dot_general(w_hi, cnt, dims,
                            preferred_element_type=jnp.float32)
        + jax.lax.dot_general(w_lo, cnt, dims,
                              preferred_element_type=jnp.float32)
    )
    s_cnt = sums[0:1, :]
    s_cx = sums[1:2, :]
    s_cy = sums[2:3, :]
    s_lw = sums[3:4, :]
    s_lh = sums[4:5, :]
    b_cx = bx_ref[0]
    b_cy = by_ref[0]
    b_w = bw_ref[0]
    b_h = bh_ref[0]
    l0 = (s_cx - b_cx * s_cnt) / b_w
    l1 = (s_cy - b_cy * s_cnt) / b_h
    l2 = s_lw - s_cnt * jnp.log(b_w)
    l3 = s_lh - s_cnt * jnp.log(b_h)

    bg = (~matched).astype(jnp.float32)
    ignore_any = jnp.max(ign.astype(jnp.int32), axis=0, keepdims=True) > 0
    amask = jnp.where(ignore_any, -1.0, bg)

    zeros2 = jnp.zeros((2, blk), jnp.float32)
    packed = jnp.concatenate(
        [cls_true, l0, l1, l2, l3, amask, zeros2], axis=0)
    packed_ref[0] = jnp.transpose(packed)


def _emit_kernel(packed_ref, cls_ref, loc_ref, msk_ref):
    d = packed_ref[0]
    cls_true = d[:, 0:1].astype(jnp.int32)
    c_iota = jax.lax.broadcasted_iota(jnp.int32, (d.shape[0], NC), 1)
    cls_ref[0] = (c_iota == cls_true).astype(jnp.float32)
    loc_ref[0] = d[:, 1:5]
    msk_ref[0] = d[:, 5:6]


@jax.jit
def kernel(gt_boxes, pr_boxes):
    B, NG, _ = gt_boxes.shape
    _, NP, _ = pr_boxes.shape
    blk = 4096
    npad = -NP % blk
    NPP = NP + npad
    pb_steps = NPP // blk

    gt_valid = gt_boxes[:, :, 0:1] != -1.0
    gt_clean = jnp.where(gt_valid, gt_boxes, jnp.zeros((), jnp.float32))
    gt_clean = jnp.concatenate([gt_clean[:, :, :5], gt_boxes[:, :, 5:6]],
                               axis=-1)

    comp = jnp.pad(pr_boxes, ((0, 0), (0, npad), (0, 0)))
    px = comp[:, :, 0].reshape(B, 1, NPP)
    py = comp[:, :, 1].reshape(B, 1, NPP)
    pw = comp[:, :, 2].reshape(B, 1, NPP)
    ph = comp[:, :, 3].reshape(B, 1, NPP)

    full_spec = lambda: pl.BlockSpec((1, 1, NPP), lambda b: (0, 0, 0))
    row_spec = lambda f: pl.BlockSpec((1, 1, blk), f)
    b0 = lambda b, p: (0, 0, p)
    bb = lambda b, p: (b, 0, p)

    best = pl.pallas_call(
        functools.partial(_argmax_kernel, num_pr=NPP),
        grid=(B,),
        in_specs=[
            pl.BlockSpec((1, NG, 6), lambda b: (b, 0, 0)),
            full_spec(), full_spec(), full_spec(), full_spec(),
        ],
        out_specs=pl.BlockSpec((1, NG, 1), lambda b: (b, 0, 0)),
        out_shape=jax.ShapeDtypeStruct((B, NG, 1), jnp.int32),
    )(gt_clean, px, py, pw, ph)

    packed = pl.pallas_call(
        functools.partial(_assign_kernel, blk=blk),
        grid=(B, pb_steps),
        in_specs=[
            pl.BlockSpec((1, NG, 6), lambda b, p: (b, 0, 0)),
            row_spec(b0), row_spec(b0), row_spec(b0), row_spec(b0),
            row_spec(bb), row_spec(bb), row_spec(bb), row_spec(bb),
            pl.BlockSpec((1, NG, 1), lambda b, p: (b, 0, 0)),
        ],
        out_specs=pl.BlockSpec((1, blk, 8), lambda b, p: (b, p, 0)),
        out_shape=jax.ShapeDtypeStruct((B, NPP, 8), jnp.float32),
    )(gt_clean, px, py, pw, ph, px, py, pw, ph, best)

    blk3 = 10000
    cls_out, loc_true, amask = pl.pallas_call(
        _emit_kernel,
        grid=(B, NP // blk3),
        in_specs=[pl.BlockSpec((1, blk3, 8), lambda b, p: (b, p, 0))],
        out_specs=[
            pl.BlockSpec((1, blk3, NC), lambda b, p: (b, p, 0)),
            pl.BlockSpec((1, blk3, 4), lambda b, p: (b, p, 0)),
            pl.BlockSpec((1, blk3, 1), lambda b, p: (b, p, 0)),
        ],
        out_shape=[
            jax.ShapeDtypeStruct((B, NP, NC), jnp.float32),
            jax.ShapeDtypeStruct((B, NP, 4), jnp.float32),
            jax.ShapeDtypeStruct((B, NP, 1), jnp.float32),
        ],
    )(packed)

    return (cls_out, loc_true, amask)


# speedup vs baseline: 1.2488x; 1.0241x over previous
import functools

import jax
import jax.numpy as jnp
from jax.experimental import pallas as pl
from jax.experimental.pallas import tpu as pltpu

NC = 80


def _corners(cx, cy, w, h):
    x1 = cx - w / 2
    y1 = cy - h / 2
    x2 = cx + w / 2
    y2 = cy + h / 2
    return y1, x1, y2, x2


def _inter_ue(g_cx, g_cy, g_w, g_h, p_cx, p_cy, p_w, p_h):
    gy1, gx1, gy2, gx2 = _corners(g_cx, g_cy, g_w, g_h)
    py1, px1, py2, px2 = _corners(p_cx, p_cy, p_w, p_h)
    in_ymin = jnp.maximum(gy1, py1)
    in_xmin = jnp.maximum(gx1, px1)
    in_ymax = jnp.minimum(gy2, py2)
    in_xmax = jnp.minimum(gx2, px2)
    in_w = jnp.maximum(0.0, in_xmax - in_xmin)
    in_h = jnp.maximum(0.0, in_ymax - in_ymin)
    inter = in_w * in_h
    pa_eps = p_w * p_h + 1e-5
    ue = ((g_w * g_h) + pa_eps) - inter
    return inter, ue


def _split_gt(gt):
    g_cx = gt[:, 0:1]
    g_cy = gt[:, 1:2]
    g_w = gt[:, 2:3]
    g_h = gt[:, 3:4]
    return g_cx, g_cy, g_w, g_h


def _argmax_kernel(gt_ref, px_ref, py_ref, pw_ref, ph_ref, best_ref, *,
                   num_pr):
    gt = gt_ref[0]
    g_cx, g_cy, g_w, g_h = _split_gt(gt)
    inter, ue = _inter_ue(g_cx, g_cy, g_w, g_h,
                          px_ref[0], py_ref[0], pw_ref[0], ph_ref[0])
    iou = inter / ue
    bmax = jnp.max(iou, axis=1, keepdims=True)
    pidx = jax.lax.broadcasted_iota(jnp.int32, iou.shape, 1)
    barg = jnp.min(jnp.where(iou == bmax, pidx, num_pr), axis=1, keepdims=True)
    best_ref[0] = barg


def _assign_kernel(gt_ref, px_ref, py_ref, pw_ref, ph_ref,
                   bx_ref, by_ref, bw_ref, bh_ref, best_ref, packed_ref,
                   *, blk):
    pb = pl.program_id(1)
    gt = gt_ref[0]
    ng = gt.shape[0]
    g_cx, g_cy, g_w, g_h = _split_gt(gt)
    g_cls = gt[:, 4:5]
    g_conf = gt[:, 5:6]

    inter, ue = _inter_ue(g_cx, g_cy, g_w, g_h,
                          px_ref[0], py_ref[0], pw_ref[0], ph_ref[0])
    thr = (inter + inter) >= ue
    ign = (2.5 * inter >= ue) & (~thr)

    pidx = jax.lax.broadcasted_iota(jnp.int32, (1, blk), 1) + pb * blk
    best = best_ref[0]
    best_x = jnp.where(g_conf > 0.0, best, -7)
    is_best = pidx == best_x

    g_iota = jax.lax.broadcasted_iota(jnp.int32, (ng, 1), 0)
    cls_i = g_cls.astype(jnp.int32)
    v_thr = g_iota * 128 + cls_i
    v_best = v_thr + ng * 128
    score = jnp.where(is_best, v_best, jnp.where(thr, v_thr, -1))
    smax = jnp.max(score, axis=0, keepdims=True)
    matched = smax >= 0
    cls_true = jnp.where(matched, smax & 127, NC).astype(jnp.float32)

    cnt = (thr.astype(jnp.bfloat16) + is_best.astype(jnp.bfloat16))
    lgw = jnp.log(jnp.where(g_w > 0.0, g_w, 1.0))
    lgh = jnp.log(jnp.where(g_h > 0.0, g_h, 1.0))
    wmat = jnp.transpose(jnp.concatenate(
        [jnp.ones_like(g_cx), g_cx, g_cy, lgw, lgh], axis=1))
    w_hi = wmat.astype(jnp.bfloat16)
    w_lo = (wmat - w_hi.astype(jnp.float32)).astype(jnp.bfloat16)
    dims = (((1,), (0,)), ((), ()))
    sums = (
        jax.lax.dot_general(w_hi, cnt, dims,
                            preferred_element_type=jnp.float32)
        + jax.lax.dot_general(w_lo, cnt, dims,
                              preferred_element_type=jnp.float32)
    )
    s_cnt = sums[0:1, :]
    s_cx = sums[1:2, :]
    s_cy = sums[2:3, :]
    s_lw = sums[3:4, :]
    s_lh = sums[4:5, :]
    b_cx = bx_ref[0]
    b_cy = by_ref[0]
    b_w = bw_ref[0]
    b_h = bh_ref[0]
    l0 = (s_cx - b_cx * s_cnt) / b_w
    l1 = (s_cy - b_cy * s_cnt) / b_h
    l2 = s_lw - s_cnt * jnp.log(b_w)
    l3 = s_lh - s_cnt * jnp.log(b_h)

    bg = (~matched).astype(jnp.float32)
    ignore_any = jnp.max(ign.astype(jnp.int32), axis=0, keepdims=True) > 0
    amask = jnp.where(ignore_any, -1.0, bg)

    zeros2 = jnp.zeros((2, blk), jnp.float32)
    packed = jnp.concatenate(
        [cls_true, l0, l1, l2, l3, amask, zeros2], axis=0)
    packed_ref[0] = jnp.transpose(packed)


def _emit_kernel(packed_ref, cls_ref, loc_ref, msk_ref):
    d = packed_ref[0]
    cls_true = d[:, 0:1].astype(jnp.int32)
    c_iota = jax.lax.broadcasted_iota(jnp.int32, (d.shape[0], NC), 1)
    cls_ref[0] = (c_iota == cls_true).astype(jnp.float32)
    loc_ref[0] = d[:, 1:5]
    msk_ref[0] = d[:, 5:6]


@jax.jit
def kernel(gt_boxes, pr_boxes):
    B, NG, _ = gt_boxes.shape
    _, NP, _ = pr_boxes.shape
    blk = 10240
    npad = -NP % blk
    NPP = NP + npad
    pb_steps = NPP // blk

    gt_valid = gt_boxes[:, :, 0:1] != -1.0
    gt_clean = jnp.where(gt_valid, gt_boxes, jnp.zeros((), jnp.float32))
    gt_clean = jnp.concatenate([gt_clean[:, :, :5], gt_boxes[:, :, 5:6]],
                               axis=-1)

    comp = jnp.pad(pr_boxes, ((0, 0), (0, npad), (0, 0)))
    px = comp[:, :, 0].reshape(B, 1, NPP)
    py = comp[:, :, 1].reshape(B, 1, NPP)
    pw = comp[:, :, 2].reshape(B, 1, NPP)
    ph = comp[:, :, 3].reshape(B, 1, NPP)

    full_spec = lambda: pl.BlockSpec((1, 1, NPP), lambda b: (0, 0, 0))
    row_spec = lambda f: pl.BlockSpec((1, 1, blk), f)
    b0 = lambda b, p: (0, 0, p)
    bb = lambda b, p: (b, 0, p)

    best = pl.pallas_call(
        functools.partial(_argmax_kernel, num_pr=NPP),
        grid=(B,),
        in_specs=[
            pl.BlockSpec((1, NG, 6), lambda b: (b, 0, 0)),
            full_spec(), full_spec(), full_spec(), full_spec(),
        ],
        out_specs=pl.BlockSpec((1, NG, 1), lambda b: (b, 0, 0)),
        out_shape=jax.ShapeDtypeStruct((B, NG, 1), jnp.int32),
    )(gt_clean, px, py, pw, ph)

    packed = pl.pallas_call(
        functools.partial(_assign_kernel, blk=blk),
        grid=(B, pb_steps),
        in_specs=[
            pl.BlockSpec((1, NG, 6), lambda b, p: (b, 0, 0)),
            row_spec(b0), row_spec(b0), row_spec(b0), row_spec(b0),
            row_spec(bb), row_spec(bb), row_spec(bb), row_spec(bb),
            pl.BlockSpec((1, NG, 1), lambda b, p: (b, 0, 0)),
        ],
        out_specs=pl.BlockSpec((1, blk, 8), lambda b, p: (b, p, 0)),
        out_shape=jax.ShapeDtypeStruct((B, NPP, 8), jnp.float32),
    )(gt_clean, px, py, pw, ph, px, py, pw, ph, best)

    blk3 = 10000
    cls_out, loc_true, amask = pl.pallas_call(
        _emit_kernel,
        grid=(B, NP // blk3),
        in_specs=[pl.BlockSpec((1, blk3, 8), lambda b, p: (b, p, 0))],
        out_specs=[
            pl.BlockSpec((1, blk3, NC), lambda b, p: (b, p, 0)),
            pl.BlockSpec((1, blk3, 4), lambda b, p: (b, p, 0)),
            pl.BlockSpec((1, blk3, 1), lambda b, p: (b, p, 0)),
        ],
        out_shape=[
            jax.ShapeDtypeStruct((B, NP, NC), jnp.float32),
            jax.ShapeDtypeStruct((B, NP, 4), jnp.float32),
            jax.ShapeDtypeStruct((B, NP, 1), jnp.float32),
        ],
    )(packed)

    return (cls_out, loc_true, amask)
